# baseline (speedup 1.0000x reference)
import functools

import jax
import jax.numpy as jnp
import numpy as np
from jax import lax
from jax.experimental import pallas as pl
from jax.experimental.pallas import tpu as pltpu
from jax.experimental.pallas import tpu_sc as plsc

_L = 16
_NC = 2
_NS = 16
_NW = _NC * _NS
_CH = 800
_GB = 128
_R = 129
_VY = 132
_VZ = 130 * _VY


def _cell_coord(v):
    ix = ((v + 1.0) * 256.0 - 1.0) * 0.5
    li = ix.astype(jnp.int32)
    fr = ix - li.astype(jnp.float32)
    return li - 127, fr


def _lo16(w):
    return plsc.bitcast(lax.shift_left(w, 16), jnp.float32)


def _hi16(w):
    return plsc.bitcast(jnp.bitwise_and(w, jnp.int32(-65536)), jnp.float32)


@functools.lru_cache(maxsize=None)
def _make_sc_kernel(m2, chunks):
    mesh = plsc.VectorSubcoreMesh(core_axis_name="c", subcore_axis_name="s")

    def body(pts, tbl, out, pts_v, idx_v, rows_v, out_v, sem):
        wid = lax.axis_index("s") * _NC + lax.axis_index("c")
        iota = lax.iota(jnp.int32, _L)
        c0 = jnp.zeros((_L,), jnp.int32)
        c1 = c0 + 1
        c2 = c0 + 2

        def coords(i):
            rows = iota + i * _L
            xv = plsc.load_gather(pts_v, [rows, c0])
            yv = plsc.load_gather(pts_v, [rows, c1])
            zv = plsc.load_gather(pts_v, [rows, c2])
            return rows, xv, yv, zv

        for t in range(chunks):
            off = (wid * chunks + t) * _CH
            pltpu.sync_copy(pts.at[pl.ds(off, _CH)], pts_v)

            def idx_body(i, carry):
                base = i * _L
                _, xv, yv, zv = coords(i)
                lx, _ = _cell_coord(xv)
                ly, _ = _cell_coord(yv)
                lz, _ = _cell_coord(zv)
                idx = (lz * 130 + ly) * _VY + lx
                idx_v[pl.ds(base, _L)] = lax.shift_right_logical(idx, 2)
                idx_v[pl.ds(_CH + base, _L)] = (
                    lax.shift_right_logical(idx + _VY, 2))
                idx_v[pl.ds(2 * _CH + base, _L)] = (
                    lax.shift_right_logical(idx + _VZ, 2))
                idx_v[pl.ds(3 * _CH + base, _L)] = (
                    lax.shift_right_logical(idx + (_VZ + _VY), 2))
                return carry

            lax.fori_loop(0, _CH // _L, idx_body, 0)

            copies = []
            for g in range(4 * _CH // _GB):
                copies.append(pltpu.async_copy(
                    tbl.at[idx_v.at[pl.ds(g * _GB, _GB)]],
                    rows_v.at[pl.ds(g * _GB, _GB)], sem))
            for cpy in copies:
                cpy.wait()

            def comp_body(i, carry):
                rows, xv, yv, zv = coords(i)
                _, fx = _cell_coord(xv)
                _, fy = _cell_coord(yv)
                _, fz = _cell_coord(zv)
                fy0 = 1.0 - fy
                fz0 = 1.0 - fz
                wyz = (fy0 * fz0, fy * fz0, fy0 * fz, fy * fz)
                wx0 = 1.0 - fx
                wl = [wx0 * w for w in wyz]
                wh = [fx * w for w in wyz]
                lxi, _ = _cell_coord(xv)
                lyi, _ = _cell_coord(yv)
                lzi, _ = _cell_coord(zv)
                vbase = (lzi * 130 + lyi) * _VY + lxi
                offs = (0, _VY, _VZ, _VZ + _VY)
                accs = [xv, yv, zv]
                for q in range(4):
                    rq = rows + (q * _CH)
                    colb = lax.shift_left(
                        jnp.bitwise_and(vbase + offs[q], jnp.int32(3)), 2)
                    for c in range(3):
                        w = plsc.load_gather(rows_v, [rq, colb + c])
                        accs[c] = accs[c] + wl[q] * _lo16(w) + wh[q] * _hi16(w)
                for c in range(3):
                    plsc.store_scatter(out_v, [rows, c0 + c], accs[c])
                return carry

            lax.fori_loop(0, _CH // _L, comp_body, 0)

            pltpu.sync_copy(out_v, out.at[pl.ds(off, _CH)])

    return pl.kernel(
        body,
        out_type=jax.ShapeDtypeStruct((m2, 3), jnp.float32),
        mesh=mesh,
        compiler_params=pltpu.CompilerParams(
            needs_layout_passes=False, use_tc_tiling_on_sc=False),
        scratch_types=[
            pltpu.VMEM((_CH, 3), jnp.float32),
            pltpu.VMEM((4 * _CH,), jnp.int32),
            pltpu.VMEM((4 * _CH, 16), jnp.int32),
            pltpu.VMEM((_CH, 3), jnp.float32),
            pltpu.SemaphoreType.DMA,
        ],
    )


_NLINE = 10
_NBATCH = 13
_SRCW = 144
_XOFF = 15
_LR = 33
_BR = _NLINE * _LR


def _round_bf16_pair(lo_bits, hi_bits):
    rlo = lax.shift_right_logical(lo_bits + jnp.int32(0x8000), 16)
    rhi = lax.shift_right_logical(hi_bits + jnp.int32(0x8000), 16)
    return rlo | lax.shift_left(rhi, 16)


def _make_build_kernel():
    mesh = plsc.VectorSubcoreMesh(core_axis_name="c", subcore_axis_name="s")
    nrows = 130 * 130 * _LR

    def body(g2d, tbl, src0_v, src1_v, src2_v, out_v, sem):
        wid = lax.axis_index("s") * _NC + lax.axis_index("c")
        srcs = (src0_v, src1_v, src2_v)
        iota = lax.iota(jnp.int32, _L)
        zeros = jnp.zeros((_L,), jnp.int32)
        lane0 = iota == 0
        rcon = [[lax.shift_right_logical(iota * 4 + (64 * g + c), 4)
                 for c in range(4)] for g in range(8)]
        ccon = [[jnp.bitwise_and(iota * 4 + (64 * g + c), jnp.int32(15))
                 for c in range(4)] for g in range(8)]

        def do_plane(p):
            for c in range(3):
                row0 = (c * 256 + 127 + p) * 256 + 127
                pltpu.sync_copy(
                    g2d.at[pl.ds(row0, _R), pl.ds(112, _SRCW)], srcs[c])

            def batch_body(b, carry):
                def line_body(yl, carry2):
                    y = b * _NLINE + yl
                    rbase = yl * _LR

                    @pl.when(y <= 128)
                    def _():
                        yv = jnp.broadcast_to(y, (_L,)).astype(jnp.int32)
                        for c in range(3):
                            sv = srcs[c]
                            for g in range(8):
                                col = _XOFF + 16 * g
                                lo = plsc.bitcast(plsc.load_gather(
                                    sv, [yv, iota + col]), jnp.int32)
                                hi = plsc.bitcast(plsc.load_gather(
                                    sv, [yv, iota + (col + 1)]), jnp.int32)
                                w = _round_bf16_pair(lo, hi)
                                plsc.store_scatter(
                                    out_v, [rcon[g][c] + rbase, ccon[g][c]],
                                    w)
                            lo = plsc.bitcast(plsc.load_gather(
                                sv, [yv, iota + (_XOFF + 128)],
                                mask=lane0), jnp.int32)
                            w = _round_bf16_pair(lo, zeros)
                            plsc.store_scatter(
                                out_v, [iota + (rbase + 32), zeros + c], w,
                                mask=lane0)
                        for g in range(8):
                            plsc.store_scatter(
                                out_v, [rcon[g][3] + rbase, ccon[g][3]],
                                zeros)
                        plsc.store_scatter(
                            out_v, [iota + (rbase + 32), zeros + 3], zeros,
                            mask=lane0)
                        plsc.store_scatter(
                            out_v, [zeros + (rbase + 32), iota], zeros,
                            mask=iota >= 4)

                    @pl.when(y == 129)
                    def _():
                        for r in range(_LR):
                            out_v[rbase + r, :] = zeros
                    return carry2

                lax.fori_loop(0, _NLINE, line_body, 0)
                roff = (p * 130 + b * _NLINE) * _LR
                pltpu.sync_copy(out_v, tbl.at[pl.ds(roff, _BR)])
                return carry

            lax.fori_loop(0, _NBATCH, batch_body, 0)

        def zero_plane(p):
            def zb(r, carry):
                out_v[r, :] = zeros
                return carry
            lax.fori_loop(0, _BR, zb, 0)

            def zdma(b, carry):
                roff = (p * 130 + b * _NLINE) * _LR
                pltpu.sync_copy(out_v, tbl.at[pl.ds(roff, _BR)])
                return carry

            lax.fori_loop(0, _NBATCH, zdma, 0)

        def plane_body(s, carry):
            p = wid + 32 * s

            @pl.when(p <= 128)
            def _():
                do_plane(p)

            @pl.when(p == 129)
            def _():
                zero_plane(p)
            return carry

        lax.fori_loop(0, 5, plane_body, 0)

    return pl.kernel(
        body,
        out_type=jax.ShapeDtypeStruct((nrows, 16), jnp.int32),
        mesh=mesh,
        compiler_params=pltpu.CompilerParams(
            needs_layout_passes=False, use_tc_tiling_on_sc=False),
        scratch_types=[
            pltpu.VMEM((_R, _SRCW), jnp.float32),
            pltpu.VMEM((_R, _SRCW), jnp.float32),
            pltpu.VMEM((_R, _SRCW), jnp.float32),
            pltpu.VMEM((_BR, 16), jnp.int32),
            pltpu.SemaphoreType.DMA,
        ],
    )


def _build_table(grid):
    g2d = grid.reshape(3 * 256 * 256, 256)
    return _make_build_kernel()(g2d)


def kernel(x, grid):
    B, N, _ = x.shape
    M = B * N
    tile = _NW * _CH
    m2 = ((M + tile - 1) // tile) * tile
    chunks = m2 // tile
    pts = jnp.pad(x.reshape(M, 3), ((0, m2 - M), (0, 0)),
                  constant_values=0.5)
    tbl = _build_table(grid)
    res = _make_sc_kernel(m2, chunks)(pts, tbl)
    return res[:M].reshape(B, N, 3)


# speedup vs baseline: 1.3158x; 1.3158x over previous
import functools

import jax
import jax.numpy as jnp
import numpy as np
from jax import lax
from jax.experimental import pallas as pl
from jax.experimental.pallas import tpu as pltpu
from jax.experimental.pallas import tpu_sc as plsc

_L = 16
_NC = 2
_NS = 16
_NW = _NC * _NS
_CH = 1024
_GB = 128
_R = 129
_VY = 132
_VZ = 130 * _VY


def _cell_coord(v):
    ix = ((v + 1.0) * 256.0 - 1.0) * 0.5
    li = ix.astype(jnp.int32)
    fr = ix - li.astype(jnp.float32)
    return li - 127, fr


def _lo16(w):
    return plsc.bitcast(lax.shift_left(w, 16), jnp.float32)


def _hi16(w):
    return plsc.bitcast(jnp.bitwise_and(w, jnp.int32(-65536)), jnp.float32)


@functools.lru_cache(maxsize=None)
def _make_sc_kernel(m):
    mesh = plsc.VectorSubcoreMesh(core_axis_name="c", subcore_axis_name="s")
    pw = (m // (_NW * _L)) * _L
    extra = m - _NW * pw
    nfull = pw // _CH
    tail = pw - nfull * _CH

    def body(pts, tbl, out, pts_v, idx_v, rows_v, out_v, sem):
        wid = lax.axis_index("s") * _NC + lax.axis_index("c")
        iota = lax.iota(jnp.int32, _L)
        c0 = jnp.zeros((_L,), jnp.int32)
        c1 = c0 + 1
        c2 = c0 + 2

        def coords(i):
            rows = iota + i * _L
            xv = plsc.load_gather(pts_v, [rows, c0])
            yv = plsc.load_gather(pts_v, [rows, c1])
            zv = plsc.load_gather(pts_v, [rows, c2])
            return rows, xv, yv, zv

        def chunk(off, n, gb):
            pltpu.sync_copy(pts.at[pl.ds(off, n)], pts_v.at[pl.ds(0, n)])

            def idx_body(i, carry):
                base = i * _L
                _, xv, yv, zv = coords(i)
                lx, _ = _cell_coord(xv)
                ly, _ = _cell_coord(yv)
                lz, _ = _cell_coord(zv)
                idx = (lz * 130 + ly) * _VY + lx
                idx_v[pl.ds(base, _L)] = lax.shift_right_logical(idx, 2)
                idx_v[pl.ds(n + base, _L)] = (
                    lax.shift_right_logical(idx + _VY, 2))
                idx_v[pl.ds(2 * n + base, _L)] = (
                    lax.shift_right_logical(idx + _VZ, 2))
                idx_v[pl.ds(3 * n + base, _L)] = (
                    lax.shift_right_logical(idx + (_VZ + _VY), 2))
                return carry

            lax.fori_loop(0, n // _L, idx_body, 0)

            copies = []
            for g in range(4 * n // gb):
                copies.append(pltpu.async_copy(
                    tbl.at[idx_v.at[pl.ds(g * gb, gb)]],
                    rows_v.at[pl.ds(g * gb, gb)], sem))
            for cpy in copies:
                cpy.wait()

            def comp_body(i, carry):
                rows, xv, yv, zv = coords(i)
                _, fx = _cell_coord(xv)
                _, fy = _cell_coord(yv)
                _, fz = _cell_coord(zv)
                fy0 = 1.0 - fy
                fz0 = 1.0 - fz
                wyz = (fy0 * fz0, fy * fz0, fy0 * fz, fy * fz)
                wx0 = 1.0 - fx
                wl = [wx0 * w for w in wyz]
                wh = [fx * w for w in wyz]
                lxi, _ = _cell_coord(xv)
                lyi, _ = _cell_coord(yv)
                lzi, _ = _cell_coord(zv)
                vbase = (lzi * 130 + lyi) * _VY + lxi
                offs = (0, _VY, _VZ, _VZ + _VY)
                accs = [xv, yv, zv]
                for q in range(4):
                    rq = rows + (q * n)
                    colb = lax.shift_left(
                        jnp.bitwise_and(vbase + offs[q], jnp.int32(3)), 2)
                    for c in range(3):
                        w = plsc.load_gather(rows_v, [rq, colb + c])
                        accs[c] = accs[c] + wl[q] * _lo16(w) + wh[q] * _hi16(w)
                for c in range(3):
                    plsc.store_scatter(out_v, [rows, c0 + c], accs[c])
                return carry

            lax.fori_loop(0, n // _L, comp_body, 0)

            pltpu.sync_copy(out_v.at[pl.ds(0, n)], out.at[pl.ds(off, n)])

        base = wid * pw
        for t in range(nfull):
            chunk(base + t * _CH, _CH, _GB)
        if tail:
            chunk(base + nfull * _CH, tail, _L)
        if extra:
            @pl.when(wid == _NW - 1)
            def _():
                chunk(_NW * pw, extra, _L)

    return pl.kernel(
        body,
        out_type=jax.ShapeDtypeStruct((m, 3), jnp.float32),
        mesh=mesh,
        compiler_params=pltpu.CompilerParams(
            needs_layout_passes=False, use_tc_tiling_on_sc=False),
        scratch_types=[
            pltpu.VMEM((_CH, 3), jnp.float32),
            pltpu.VMEM((4 * _CH,), jnp.int32),
            pltpu.VMEM((4 * _CH, 16), jnp.int32),
            pltpu.VMEM((_CH, 3), jnp.float32),
            pltpu.SemaphoreType.DMA,
        ],
    )


_NLINE = 10
_NBATCH = 13
_SRCW = 144
_XOFF = 15
_LR = 33
_BR = _NLINE * _LR


def _round_bf16_pair(lo_bits, hi_bits):
    rlo = lax.shift_right_logical(lo_bits + jnp.int32(0x8000), 16)
    rhi = lax.shift_right_logical(hi_bits + jnp.int32(0x8000), 16)
    return rlo | lax.shift_left(rhi, 16)


def _make_build_kernel():
    mesh = plsc.VectorSubcoreMesh(core_axis_name="c", subcore_axis_name="s")
    nrows = 130 * 130 * _LR

    def body(g2d, tbl, src0_v, src1_v, src2_v, out_v, sem):
        wid = lax.axis_index("s") * _NC + lax.axis_index("c")
        srcs = (src0_v, src1_v, src2_v)
        iota = lax.iota(jnp.int32, _L)
        zeros = jnp.zeros((_L,), jnp.int32)
        lane0 = iota == 0
        rcon = [[lax.shift_right_logical(iota * 4 + (64 * g + c), 4)
                 for c in range(4)] for g in range(8)]
        ccon = [[jnp.bitwise_and(iota * 4 + (64 * g + c), jnp.int32(15))
                 for c in range(4)] for g in range(8)]

        def do_plane(p):
            for c in range(3):
                row0 = (c * 256 + 127 + p) * 256 + 127
                pltpu.sync_copy(
                    g2d.at[pl.ds(row0, _R), pl.ds(112, _SRCW)], srcs[c])

            def batch_body(b, carry):
                def line_body(yl, carry2):
                    y = b * _NLINE + yl
                    rbase = yl * _LR

                    @pl.when(y <= 128)
                    def _():
                        yv = jnp.broadcast_to(y, (_L,)).astype(jnp.int32)
                        for c in range(3):
                            sv = srcs[c]
                            for g in range(8):
                                col = _XOFF + 16 * g
                                lo = plsc.bitcast(plsc.load_gather(
                                    sv, [yv, iota + col]), jnp.int32)
                                hi = plsc.bitcast(plsc.load_gather(
                                    sv, [yv, iota + (col + 1)]), jnp.int32)
                                w = _round_bf16_pair(lo, hi)
                                plsc.store_scatter(
                                    out_v, [rcon[g][c] + rbase, ccon[g][c]],
                                    w)
                            lo = plsc.bitcast(plsc.load_gather(
                                sv, [yv, iota + (_XOFF + 128)],
                                mask=lane0), jnp.int32)
                            w = _round_bf16_pair(lo, zeros)
                            plsc.store_scatter(
                                out_v, [iota + (rbase + 32), zeros + c], w,
                                mask=lane0)
                        for g in range(8):
                            plsc.store_scatter(
                                out_v, [rcon[g][3] + rbase, ccon[g][3]],
                                zeros)
                        plsc.store_scatter(
                            out_v, [iota + (rbase + 32), zeros + 3], zeros,
                            mask=lane0)
                        plsc.store_scatter(
                            out_v, [zeros + (rbase + 32), iota], zeros,
                            mask=iota >= 4)

                    @pl.when(y == 129)
                    def _():
                        for r in range(_LR):
                            out_v[rbase + r, :] = zeros
                    return carry2

                lax.fori_loop(0, _NLINE, line_body, 0)
                roff = (p * 130 + b * _NLINE) * _LR
                pltpu.sync_copy(out_v, tbl.at[pl.ds(roff, _BR)])
                return carry

            lax.fori_loop(0, _NBATCH, batch_body, 0)

        def zero_plane(p):
            def zb(r, carry):
                out_v[r, :] = zeros
                return carry
            lax.fori_loop(0, _BR, zb, 0)

            def zdma(b, carry):
                roff = (p * 130 + b * _NLINE) * _LR
                pltpu.sync_copy(out_v, tbl.at[pl.ds(roff, _BR)])
                return carry

            lax.fori_loop(0, _NBATCH, zdma, 0)

        def plane_body(s, carry):
            p = wid + 32 * s

            @pl.when(p <= 128)
            def _():
                do_plane(p)

            @pl.when(p == 129)
            def _():
                zero_plane(p)
            return carry

        lax.fori_loop(0, 5, plane_body, 0)

    return pl.kernel(
        body,
        out_type=jax.ShapeDtypeStruct((nrows, 16), jnp.int32),
        mesh=mesh,
        compiler_params=pltpu.CompilerParams(
            needs_layout_passes=False, use_tc_tiling_on_sc=False),
        scratch_types=[
            pltpu.VMEM((_R, _SRCW), jnp.float32),
            pltpu.VMEM((_R, _SRCW), jnp.float32),
            pltpu.VMEM((_R, _SRCW), jnp.float32),
            pltpu.VMEM((_BR, 16), jnp.int32),
            pltpu.SemaphoreType.DMA,
        ],
    )


def _build_table(grid):
    g2d = grid.reshape(3 * 256 * 256, 256)
    return _make_build_kernel()(g2d)


def kernel(x, grid):
    B, N, _ = x.shape
    M = B * N
    tbl = _build_table(grid)
    res = _make_sc_kernel(M)(x.reshape(M, 3), tbl)
    return res.reshape(B, N, 3)


# speedup vs baseline: 2.5452x; 1.9343x over previous
import functools

import jax
import jax.numpy as jnp
import numpy as np
from jax import lax
from jax.experimental import pallas as pl
from jax.experimental.pallas import tpu as pltpu
from jax.experimental.pallas import tpu_sc as plsc

_L = 16
_NC = 2
_NS = 16
_NW = _NC * _NS
_CH = 1024
_GB = 128
_R = 129
_VY = 132
_VZ = 130 * _VY


def _cell_coord(v):
    ix = ((v + 1.0) * 256.0 - 1.0) * 0.5
    li = ix.astype(jnp.int32)
    fr = ix - li.astype(jnp.float32)
    return li - 127, fr


def _lo16(w):
    return plsc.bitcast(lax.shift_left(w, 16), jnp.float32)


def _hi16(w):
    return plsc.bitcast(jnp.bitwise_and(w, jnp.int32(-65536)), jnp.float32)


@functools.lru_cache(maxsize=None)
def _make_sc_kernel(m2):
    mesh = plsc.VectorSubcoreMesh(core_axis_name="c", subcore_axis_name="s")
    pw = m2 // _NW
    nfull = pw // _CH
    tail = pw - nfull * _CH
    extra = 0

    def body(xs, ys, zs, tbl, o0, o1, o2,
             xs_v, ys_v, zs_v, idx_v, rows_v, o0_v, o1_v, o2_v, sem):
        wid = lax.axis_index("s") * _NC + lax.axis_index("c")
        iota = lax.iota(jnp.int32, _L)

        def coords(i):
            rows = iota + i * _L
            base = i * _L
            return (rows, xs_v[pl.ds(base, _L)], ys_v[pl.ds(base, _L)],
                    zs_v[pl.ds(base, _L)])

        def chunk(off, n, gb):
            pltpu.sync_copy(xs.at[pl.ds(off, n)], xs_v.at[pl.ds(0, n)])
            pltpu.sync_copy(ys.at[pl.ds(off, n)], ys_v.at[pl.ds(0, n)])
            pltpu.sync_copy(zs.at[pl.ds(off, n)], zs_v.at[pl.ds(0, n)])

            def idx_body(i, carry):
                base = i * _L
                _, xv, yv, zv = coords(i)
                lx, _ = _cell_coord(xv)
                ly, _ = _cell_coord(yv)
                lz, _ = _cell_coord(zv)
                idx = (lz * 130 + ly) * _VY + lx
                idx_v[pl.ds(base, _L)] = lax.shift_right_logical(idx, 2)
                idx_v[pl.ds(n + base, _L)] = (
                    lax.shift_right_logical(idx + _VY, 2))
                idx_v[pl.ds(2 * n + base, _L)] = (
                    lax.shift_right_logical(idx + _VZ, 2))
                idx_v[pl.ds(3 * n + base, _L)] = (
                    lax.shift_right_logical(idx + (_VZ + _VY), 2))
                return carry

            lax.fori_loop(0, n // _L, idx_body, 0)

            copies = []
            for g in range(4 * n // gb):
                copies.append(pltpu.async_copy(
                    tbl.at[idx_v.at[pl.ds(g * gb, gb)]],
                    rows_v.at[pl.ds(g * gb, gb)], sem))
            for cpy in copies:
                cpy.wait()

            def comp_body(i, carry):
                rows, xv, yv, zv = coords(i)
                _, fx = _cell_coord(xv)
                _, fy = _cell_coord(yv)
                _, fz = _cell_coord(zv)
                fy0 = 1.0 - fy
                fz0 = 1.0 - fz
                wyz = (fy0 * fz0, fy * fz0, fy0 * fz, fy * fz)
                wx0 = 1.0 - fx
                wl = [wx0 * w for w in wyz]
                wh = [fx * w for w in wyz]
                lxi, _ = _cell_coord(xv)
                lyi, _ = _cell_coord(yv)
                lzi, _ = _cell_coord(zv)
                vbase = (lzi * 130 + lyi) * _VY + lxi
                offs = (0, _VY, _VZ, _VZ + _VY)
                accs = [xv, yv, zv]
                for q in range(4):
                    rq = rows + (q * n)
                    colb = lax.shift_left(
                        jnp.bitwise_and(vbase + offs[q], jnp.int32(3)), 2)
                    for c in range(3):
                        w = plsc.load_gather(rows_v, [rq, colb + c])
                        accs[c] = accs[c] + wl[q] * _lo16(w) + wh[q] * _hi16(w)
                base = i * _L
                o0_v[pl.ds(base, _L)] = accs[0]
                o1_v[pl.ds(base, _L)] = accs[1]
                o2_v[pl.ds(base, _L)] = accs[2]
                return carry

            lax.fori_loop(0, n // _L, comp_body, 0)

            pltpu.sync_copy(o0_v.at[pl.ds(0, n)], o0.at[pl.ds(off, n)])
            pltpu.sync_copy(o1_v.at[pl.ds(0, n)], o1.at[pl.ds(off, n)])
            pltpu.sync_copy(o2_v.at[pl.ds(0, n)], o2.at[pl.ds(off, n)])

        base = wid * pw
        for t in range(nfull):
            chunk(base + t * _CH, _CH, _GB)
        if tail:
            chunk(base + nfull * _CH, tail, _L)

    fvec = jax.ShapeDtypeStruct((m2,), jnp.float32)
    return pl.kernel(
        body,
        out_type=[fvec, fvec, fvec],
        mesh=mesh,
        compiler_params=pltpu.CompilerParams(
            needs_layout_passes=False, use_tc_tiling_on_sc=False),
        scratch_types=[
            pltpu.VMEM((_CH,), jnp.float32),
            pltpu.VMEM((_CH,), jnp.float32),
            pltpu.VMEM((_CH,), jnp.float32),
            pltpu.VMEM((4 * _CH,), jnp.int32),
            pltpu.VMEM((4 * _CH, 16), jnp.int32),
            pltpu.VMEM((_CH,), jnp.float32),
            pltpu.VMEM((_CH,), jnp.float32),
            pltpu.VMEM((_CH,), jnp.float32),
            pltpu.SemaphoreType.DMA,
        ],
    )


_NLINE = 10
_NBATCH = 13
_SRCW = 144
_XOFF = 15
_LR = 33
_BR = _NLINE * _LR


def _round_bf16_pair(lo_bits, hi_bits):
    rlo = lax.shift_right_logical(lo_bits + jnp.int32(0x8000), 16)
    rhi = lax.shift_right_logical(hi_bits + jnp.int32(0x8000), 16)
    return rlo | lax.shift_left(rhi, 16)


def _make_build_kernel():
    mesh = plsc.VectorSubcoreMesh(core_axis_name="c", subcore_axis_name="s")
    nrows = 130 * 130 * _LR

    def body(g2d, tbl, src0_v, src1_v, src2_v, out_v, sem):
        wid = lax.axis_index("s") * _NC + lax.axis_index("c")
        srcs = (src0_v, src1_v, src2_v)
        iota = lax.iota(jnp.int32, _L)
        zeros = jnp.zeros((_L,), jnp.int32)
        lane0 = iota == 0
        rcon = [[lax.shift_right_logical(iota * 4 + (64 * g + c), 4)
                 for c in range(4)] for g in range(8)]
        ccon = [[jnp.bitwise_and(iota * 4 + (64 * g + c), jnp.int32(15))
                 for c in range(4)] for g in range(8)]

        def do_plane(p):
            for c in range(3):
                row0 = (c * 256 + 127 + p) * 256 + 127
                pltpu.sync_copy(
                    g2d.at[pl.ds(row0, _R), pl.ds(112, _SRCW)], srcs[c])

            def batch_body(b, carry):
                def line_body(yl, carry2):
                    y = b * _NLINE + yl
                    rbase = yl * _LR

                    @pl.when(y <= 128)
                    def _():
                        yv = jnp.broadcast_to(y, (_L,)).astype(jnp.int32)
                        for c in range(3):
                            sv = srcs[c]
                            for g in range(8):
                                col = _XOFF + 16 * g
                                lo = plsc.bitcast(plsc.load_gather(
                                    sv, [yv, iota + col]), jnp.int32)
                                hi = plsc.bitcast(plsc.load_gather(
                                    sv, [yv, iota + (col + 1)]), jnp.int32)
                                w = _round_bf16_pair(lo, hi)
                                plsc.store_scatter(
                                    out_v, [rcon[g][c] + rbase, ccon[g][c]],
                                    w)
                            lo = plsc.bitcast(plsc.load_gather(
                                sv, [yv, iota + (_XOFF + 128)],
                                mask=lane0), jnp.int32)
                            w = _round_bf16_pair(lo, zeros)
                            plsc.store_scatter(
                                out_v, [iota + (rbase + 32), zeros + c], w,
                                mask=lane0)
                        for g in range(8):
                            plsc.store_scatter(
                                out_v, [rcon[g][3] + rbase, ccon[g][3]],
                                zeros)
                        plsc.store_scatter(
                            out_v, [iota + (rbase + 32), zeros + 3], zeros,
                            mask=lane0)
                        plsc.store_scatter(
                            out_v, [zeros + (rbase + 32), iota], zeros,
                            mask=iota >= 4)

                    @pl.when(y == 129)
                    def _():
                        for r in range(_LR):
                            out_v[rbase + r, :] = zeros
                    return carry2

                lax.fori_loop(0, _NLINE, line_body, 0)
                roff = (p * 130 + b * _NLINE) * _LR
                pltpu.sync_copy(out_v, tbl.at[pl.ds(roff, _BR)])
                return carry

            lax.fori_loop(0, _NBATCH, batch_body, 0)

        def zero_plane(p):
            def zb(r, carry):
                out_v[r, :] = zeros
                return carry
            lax.fori_loop(0, _BR, zb, 0)

            def zdma(b, carry):
                roff = (p * 130 + b * _NLINE) * _LR
                pltpu.sync_copy(out_v, tbl.at[pl.ds(roff, _BR)])
                return carry

            lax.fori_loop(0, _NBATCH, zdma, 0)

        def plane_body(s, carry):
            p = wid + 32 * s

            @pl.when(p <= 128)
            def _():
                do_plane(p)

            @pl.when(p == 129)
            def _():
                zero_plane(p)
            return carry

        lax.fori_loop(0, 5, plane_body, 0)

    return pl.kernel(
        body,
        out_type=jax.ShapeDtypeStruct((nrows, 16), jnp.int32),
        mesh=mesh,
        compiler_params=pltpu.CompilerParams(
            needs_layout_passes=False, use_tc_tiling_on_sc=False),
        scratch_types=[
            pltpu.VMEM((_R, _SRCW), jnp.float32),
            pltpu.VMEM((_R, _SRCW), jnp.float32),
            pltpu.VMEM((_R, _SRCW), jnp.float32),
            pltpu.VMEM((_BR, 16), jnp.int32),
            pltpu.SemaphoreType.DMA,
        ],
    )


def _build_table(grid):
    g2d = grid.reshape(3 * 256 * 256, 256)
    return _make_build_kernel()(g2d)


def kernel(x, grid):
    B, N, _ = x.shape
    M = B * N
    tile = _NW * _L
    m2 = ((M + tile - 1) // tile) * tile
    pts = x.reshape(M, 3).T
    pts = jnp.pad(pts, ((0, 0), (0, m2 - M)), constant_values=0.5)
    tbl = _build_table(grid)
    o0, o1, o2 = _make_sc_kernel(m2)(pts[0], pts[1], pts[2], tbl)
    return jnp.stack([o0[:M], o1[:M], o2[:M]], axis=-1).reshape(B, N, 3)
